# baseline (speedup 1.0000x reference)
import functools

import jax
import jax.numpy as jnp
from jax import lax
from jax.experimental import pallas as pl
from jax.experimental.pallas import tpu as pltpu
from jax.experimental.pallas import tpu_sc as plsc

B, S, D, K = 4, 8192, 1024, 8
NC, NS = 2, 16
NW = NC * NS
GROUPS = D // 16
TASKS = B * GROUPS
TPW = TASKS // NW
SB = 2048


def _sc_body(x_hbm, out_hbm, buf0, buf1, obuf, sem0, sem1):
    wid = lax.axis_index("s") * NC + lax.axis_index("c")

    bufs = (buf0, buf1)
    sems = (sem0, sem1)
    n_chunks = S // SB

    def make_row_body(buf):
        def row_body(r, Ts):
            v = buf[r, :]
            new = []
            for k in range(K):
                tk = Ts[k]
                hi = jnp.maximum(tk, v)
                v = jnp.minimum(tk, v)
                new.append(hi)
            return tuple(new)
        return row_body

    for i in range(TPW):
        t = wid * TPW + i
        b = t // GROUPS
        col = (t % GROUPS) * 16

        def copy_chunk(c):
            return pltpu.async_copy(
                x_hbm.at[b, pl.ds(c * SB, SB), pl.ds(col, 16)],
                bufs[c % 2], sems[c % 2])

        pending = copy_chunk(0)
        T = tuple(jnp.full((16,), -jnp.inf, jnp.float32) for _ in range(K))
        for c in range(n_chunks):
            pending.wait()
            if c + 1 < n_chunks:
                pending = copy_chunk(c + 1)
            T = lax.fori_loop(0, SB, make_row_body(bufs[c % 2]), T,
                              unroll=4)

        for j in range(K):
            obuf[j, :] = T[j]
        pltpu.sync_copy(obuf, out_hbm.at[b, :, pl.ds(col, 16)])


def kernel(inputs):
    out = pl.kernel(
        _sc_body,
        mesh=plsc.VectorSubcoreMesh(core_axis_name="c", subcore_axis_name="s"),
        out_type=jax.ShapeDtypeStruct((B, K, D), jnp.float32),
        compiler_params=pltpu.CompilerParams(use_tc_tiling_on_sc=False),
        scratch_types=[
            pltpu.VMEM((SB, 16), jnp.float32),
            pltpu.VMEM((SB, 16), jnp.float32),
            pltpu.VMEM((K, 16), jnp.float32),
            pltpu.SemaphoreType.DMA,
            pltpu.SemaphoreType.DMA,
        ],
    )(inputs)
    return out.transpose(0, 2, 1).reshape(B, D * K)


# speedup vs baseline: 1.8654x; 1.8654x over previous
import functools

import jax
import jax.numpy as jnp
from jax import lax
from jax.experimental import pallas as pl
from jax.experimental.pallas import tpu as pltpu
from jax.experimental.pallas import tpu_sc as plsc

B, S, D, K = 4, 8192, 1024, 8
S_SC = 1024
S_TC = S - S_SC
CHUNK = 1024
NC, NS = 2, 16
NW = NC * NS
GROUPS = D // 16
TASKS = B * GROUPS
TPW = TASKS // NW
SB = S_SC

_SORT8 = ((0, 1), (2, 3), (4, 5), (6, 7), (0, 2), (1, 3), (4, 6), (5, 7),
          (1, 2), (5, 6), (0, 4), (1, 5), (2, 6), (3, 7), (2, 4), (3, 5),
          (1, 2), (3, 4), (5, 6))
_BITONIC8 = ((0, 4), (1, 5), (2, 6), (3, 7), (0, 2), (1, 3), (4, 6), (5, 7),
             (0, 1), (2, 3), (4, 5), (6, 7))


def _ce(items, pairs):
    for i, j in pairs:
        a, b = items[i], items[j]
        items[i] = jnp.maximum(a, b)
        items[j] = jnp.minimum(a, b)


def _tc_kernel(x_ref, o_ref, t_ref, *, n_chunks):
    c = pl.program_id(1)

    @pl.when(c == 0)
    def _init():
        t_ref[...] = jnp.full(t_ref.shape, -jnp.inf, jnp.float32)

    T = [t_ref[8 * k:8 * k + 8, :] for k in range(K)]

    def group_body(g, Ts):
        a = [x_ref[0, pl.ds(g * 128 + j * 8, 8), :] for j in range(8)]
        b = [x_ref[0, pl.ds(g * 128 + 64 + j * 8, 8), :] for j in range(8)]
        _ce(a, _SORT8)
        _ce(b, _SORT8)
        s = [jnp.maximum(a[i], b[7 - i]) for i in range(8)]
        _ce(s, _BITONIC8)
        merged = [jnp.maximum(Ts[i], s[7 - i]) for i in range(8)]
        _ce(merged, _BITONIC8)
        return tuple(merged)

    T = jax.lax.fori_loop(0, CHUNK // 128, group_body, tuple(T), unroll=4)
    for k in range(K):
        t_ref[8 * k:8 * k + 8, :] = T[k]

    @pl.when(c == n_chunks - 1)
    def _finalize():
        cand = t_ref[...]
        iota = jax.lax.broadcasted_iota(jnp.int32, cand.shape, 0)
        for j in range(K):
            m = jnp.max(cand, axis=0, keepdims=True)
            idx = jnp.min(jnp.where(cand == m, iota, cand.shape[0]), axis=0,
                          keepdims=True)
            cand = jnp.where(iota == idx, -jnp.inf, cand)
            o_ref[0, j, :] = m[0]


def _tc_call(inputs):
    n_chunks = S_TC // CHUNK
    return pl.pallas_call(
        functools.partial(_tc_kernel, n_chunks=n_chunks),
        grid=(B, n_chunks),
        in_specs=[pl.BlockSpec((1, CHUNK, D), lambda b, c: (b, c, 0))],
        out_specs=pl.BlockSpec((1, K, D), lambda b, c: (b, 0, 0)),
        out_shape=jax.ShapeDtypeStruct((B, K, D), jnp.float32),
        scratch_shapes=[pltpu.VMEM((8 * K, D), jnp.float32)],
    )(inputs)


def _sc_body(x_hbm, out_hbm, buf0, buf1, obuf, sem0, sem1):
    wid = lax.axis_index("s") * NC + lax.axis_index("c")
    bufs = (buf0, buf1)
    sems = (sem0, sem1)
    n_chunks = S_SC // SB

    def make_row_body(buf):
        def row_body(r, Ts):
            v = buf[r, :]
            new = []
            for k in range(K):
                tk = Ts[k]
                hi = jnp.maximum(tk, v)
                v = jnp.minimum(tk, v)
                new.append(hi)
            return tuple(new)
        return row_body

    for i in range(TPW):
        t = wid * TPW + i
        b = t // GROUPS
        col = (t % GROUPS) * 16

        def copy_chunk(c):
            return pltpu.async_copy(
                x_hbm.at[b, pl.ds(S_TC + c * SB, SB), pl.ds(col, 16)],
                bufs[c % 2], sems[c % 2])

        pending = copy_chunk(0)
        T = tuple(jnp.full((16,), -jnp.inf, jnp.float32) for _ in range(K))
        for c in range(n_chunks):
            pending.wait()
            if c + 1 < n_chunks:
                pending = copy_chunk(c + 1)
            T = lax.fori_loop(0, SB, make_row_body(bufs[c % 2]), T,
                              unroll=4)

        for j in range(K):
            obuf[j, :] = T[j]
        pltpu.sync_copy(obuf, out_hbm.at[b, :, pl.ds(col, 16)])


def _sc_call(inputs):
    return pl.kernel(
        _sc_body,
        mesh=plsc.VectorSubcoreMesh(core_axis_name="c", subcore_axis_name="s"),
        out_type=jax.ShapeDtypeStruct((B, K, D), jnp.float32),
        compiler_params=pltpu.CompilerParams(use_tc_tiling_on_sc=False),
        scratch_types=[
            pltpu.VMEM((SB, 16), jnp.float32),
            pltpu.VMEM((SB, 16), jnp.float32),
            pltpu.VMEM((K, 16), jnp.float32),
            pltpu.SemaphoreType.DMA,
            pltpu.SemaphoreType.DMA,
        ],
    )(inputs)


def _merge_kernel(a_ref, b_ref, o_ref):
    A = [a_ref[0, i, :] for i in range(K)]
    Bv = [b_ref[0, i, :] for i in range(K)]
    C = [jnp.maximum(A[i], Bv[7 - i]) for i in range(K)]
    _ce(C, _BITONIC8)
    for i in range(K):
        o_ref[0, i, :] = C[i]


def _merge_call(a, b):
    return pl.pallas_call(
        _merge_kernel,
        grid=(B,),
        in_specs=[pl.BlockSpec((1, K, D), lambda i: (i, 0, 0)),
                  pl.BlockSpec((1, K, D), lambda i: (i, 0, 0))],
        out_specs=pl.BlockSpec((1, K, D), lambda i: (i, 0, 0)),
        out_shape=jax.ShapeDtypeStruct((B, K, D), jnp.float32),
    )(a, b)


def kernel(inputs):
    tc = _tc_call(inputs)
    sc = _sc_call(inputs)
    out = _merge_call(tc, sc)
    return out.transpose(0, 2, 1).reshape(B, D * K)


# speedup vs baseline: 5.3465x; 2.8662x over previous
import functools

import jax
import jax.numpy as jnp
from jax.experimental import pallas as pl
from jax.experimental.pallas import tpu as pltpu

K = 8
CHUNK = 2048
DB = 1024

_SORT8 = ((0, 1), (2, 3), (4, 5), (6, 7), (0, 2), (1, 3), (4, 6), (5, 7),
          (1, 2), (5, 6), (0, 4), (1, 5), (2, 6), (3, 7), (2, 4), (3, 5),
          (1, 2), (3, 4), (5, 6))
_BITONIC8 = ((0, 4), (1, 5), (2, 6), (3, 7), (0, 2), (1, 3), (4, 6), (5, 7),
             (0, 1), (2, 3), (4, 5), (6, 7))


def _ce(items, pairs):
    for i, j in pairs:
        a, b = items[i], items[j]
        items[i] = jnp.maximum(a, b)
        items[j] = jnp.minimum(a, b)


def _kmax_kernel(x_ref, o_ref, t_ref, *, n_chunks):
    c = pl.program_id(2)

    @pl.when(c == 0)
    def _init():
        t_ref[...] = jnp.full(t_ref.shape, -jnp.inf, jnp.float32)

    T = [t_ref[8 * k:8 * k + 8, :] for k in range(K)]

    def group_body(g, Ts):
        a = [x_ref[0, pl.ds(g * 128 + j * 8, 8), :] for j in range(8)]
        b = [x_ref[0, pl.ds(g * 128 + 64 + j * 8, 8), :] for j in range(8)]
        _ce(a, _SORT8)
        _ce(b, _SORT8)
        s = [jnp.maximum(a[i], b[7 - i]) for i in range(8)]
        _ce(s, _BITONIC8)
        merged = [jnp.maximum(Ts[i], s[7 - i]) for i in range(8)]
        _ce(merged, _BITONIC8)
        return tuple(merged)

    T = jax.lax.fori_loop(0, CHUNK // 128, group_body, tuple(T), unroll=8)
    for k in range(K):
        t_ref[8 * k:8 * k + 8, :] = T[k]

    @pl.when(c == n_chunks - 1)
    def _finalize():
        cand = t_ref[...]
        iota = jax.lax.broadcasted_iota(jnp.int32, cand.shape, 0)
        for j in range(K):
            m = jnp.max(cand, axis=0, keepdims=True)
            idx = jnp.min(jnp.where(cand == m, iota, cand.shape[0]), axis=0,
                          keepdims=True)
            cand = jnp.where(iota == idx, -jnp.inf, cand)
            o_ref[0, j, :] = m[0]


def kernel(inputs):
    B, S, D = inputs.shape
    n_chunks = S // CHUNK
    out = pl.pallas_call(
        functools.partial(_kmax_kernel, n_chunks=n_chunks),
        grid=(B, D // DB, n_chunks),
        in_specs=[pl.BlockSpec((1, CHUNK, DB), lambda b, d, c: (b, c, d))],
        out_specs=pl.BlockSpec((1, K, DB), lambda b, d, c: (b, 0, d)),
        out_shape=jax.ShapeDtypeStruct((B, K, D), jnp.float32),
        scratch_shapes=[pltpu.VMEM((8 * K, DB), jnp.float32)],
    )(inputs)
    return out.transpose(0, 2, 1).reshape(B, D * K)
